# baseline (speedup 1.0000x reference)
import functools

import jax
import jax.numpy as jnp
from jax import lax
from jax.experimental import pallas as pl
from jax.experimental.pallas import tpu as pltpu
from jax.experimental.pallas import tpu_sc as plsc

N = 10000
E = 320000
D = 128
HD = D // 2
NC = 2
NS = 16
NW = NC * NS
CH = 80

EW_DEG = E // NW
ITERS_DEG = EW_DEG // CH

EW = E // NS
ITERS = EW // CH

ROWS_PER_SUB = N // NS
STG = 25
NSTG = ROWS_PER_SUB // STG
DEG_CH = 624
DEG_REM = N - NS * DEG_CH

_mesh = plsc.VectorSubcoreMesh(core_axis_name="c", subcore_axis_name="s")


def _fill1d(ref, n, value):
    vec = jnp.full((16,), value, jnp.float32)

    def body(i, carry):
        ref[pl.ds(i * 16, 16)] = vec
        return carry

    lax.fori_loop(0, n // 16, body, 0)


def _fill2d(ref, rows, cols, value):
    vec = jnp.full((16,), value, jnp.float32)

    def rbody(r, carry):
        for cidx in range(cols // 16):
            ref[r, pl.ds(cidx * 16, 16)] = vec
        return carry

    lax.fori_loop(0, rows, rbody, 0)


@functools.partial(
    pl.kernel,
    out_type=jax.ShapeDtypeStruct((NC * N,), jnp.float32),
    mesh=_mesh,
    compiler_params=pltpu.CompilerParams(use_tc_tiling_on_sc=False),
    scratch_types=[
        pltpu.VMEM_SHARED((N,), jnp.float32),
        pltpu.VMEM((CH,), jnp.int32),
        pltpu.VMEM((CH,), jnp.float32),
        pltpu.VMEM((DEG_CH,), jnp.float32),
    ],
)
def _deg_kernel(col_hbm, deg_out, acc, colbuf, onesbuf, stage):
    c = lax.axis_index("c")
    s = lax.axis_index("s")
    wid = c * NS + s

    _fill1d(stage, DEG_CH, 0.0)
    _fill1d(onesbuf, CH, 1.0)
    pltpu.sync_copy(stage, acc.at[pl.ds(s * DEG_CH, DEG_CH)])

    @pl.when(s == NS - 1)
    def _zero_tail():
        pltpu.sync_copy(stage.at[pl.ds(0, DEG_REM)],
                        acc.at[pl.ds(NS * DEG_CH, DEG_REM)])

    plsc.subcore_barrier()

    def body(i, carry):
        base = wid * EW_DEG + i * CH
        pltpu.sync_copy(col_hbm.at[pl.ds(base, CH)], colbuf)
        pltpu.sync_copy(onesbuf, acc.at[colbuf], add=True)
        return carry

    lax.fori_loop(0, ITERS_DEG, body, 0)
    plsc.subcore_barrier()

    pltpu.sync_copy(acc.at[pl.ds(s * DEG_CH, DEG_CH)], stage)
    pltpu.sync_copy(stage, deg_out.at[pl.ds(c * N + s * DEG_CH, DEG_CH)])

    @pl.when(s == NS - 1)
    def _copy_tail():
        pltpu.sync_copy(acc.at[pl.ds(NS * DEG_CH, DEG_REM)],
                        stage.at[pl.ds(0, DEG_REM)])
        pltpu.sync_copy(stage.at[pl.ds(0, DEG_REM)],
                        deg_out.at[pl.ds(c * N + NS * DEG_CH, DEG_REM)])


@functools.partial(
    pl.kernel,
    out_type=jax.ShapeDtypeStruct((NC, N, HD), jnp.float32),
    mesh=_mesh,
    compiler_params=pltpu.CompilerParams(use_tc_tiling_on_sc=False),
    scratch_types=[
        pltpu.VMEM_SHARED((N, HD), jnp.float32),
        pltpu.VMEM((ITERS, CH), jnp.int32),
        pltpu.VMEM((ITERS, CH), jnp.int32),
        pltpu.VMEM((CH, HD), jnp.float32),
        pltpu.VMEM((CH, HD), jnp.float32),
        pltpu.VMEM((CH, HD), jnp.float32),
        pltpu.VMEM((CH, HD), jnp.float32),
        pltpu.VMEM((STG, HD), jnp.float32),
        pltpu.SemaphoreType.DMA,
        pltpu.SemaphoreType.DMA,
        pltpu.SemaphoreType.DMA,
        pltpu.SemaphoreType.DMA,
        pltpu.SemaphoreType.DMA,
        pltpu.SemaphoreType.DMA,
        pltpu.SemaphoreType.DMA,
        pltpu.SemaphoreType.DMA,
        pltpu.SemaphoreType.DMA,
    ],
)
def _agg_kernel(p_hbm, row_hbm, col_hbm, g_out,
                acc, rowidx, colidx, buf0, buf1, buf2, buf3, stage,
                sg0, sg1, sg2, sg3, ss0, ss1, ss2, ss3, sidx):
    c = lax.axis_index("c")
    s = lax.axis_index("s")
    bufs = (buf0, buf1, buf2, buf3)
    sgs = (sg0, sg1, sg2, sg3)
    sss = (ss0, ss1, ss2, ss3)
    phalf = p_hbm.at[c]

    di_r = pltpu.async_copy(row_hbm.at[s], rowidx, sidx)
    di_c = pltpu.async_copy(col_hbm.at[s], colidx, sidx)

    _fill2d(stage, STG, HD, 0.0)
    di_r.wait()
    di_c.wait()

    pltpu.async_copy(phalf.at[rowidx.at[0]], bufs[0], sgs[0])
    pltpu.async_copy(phalf.at[rowidx.at[1]], bufs[1], sgs[1])

    def zbody(k, carry):
        pltpu.sync_copy(stage, acc.at[pl.ds(s * ROWS_PER_SUB + k * STG, STG)])
        return carry

    lax.fori_loop(0, NSTG, zbody, 0)
    plsc.subcore_barrier()

    def chunk(k, b, wait_prev_scatter, start_next_gather):
        buf, sg, ss = bufs[b], sgs[b], sss[b]
        pltpu.make_async_copy(phalf.at[rowidx.at[k]], buf, sg).wait()
        pltpu.async_copy(buf, acc.at[colidx.at[k]], ss, add=True)
        if start_next_gather:
            b2 = (b + 2) % 4
            if wait_prev_scatter:
                pltpu.make_async_copy(bufs[b2], acc.at[colidx.at[k]],
                                      sss[b2]).wait()
            pltpu.async_copy(phalf.at[rowidx.at[k + 2]], bufs[b2], sgs[b2])

    chunk(0, 0, False, True)
    chunk(1, 1, False, True)

    def loop_body(k4, carry):
        k = 2 + 4 * k4
        chunk(k, 2, True, True)
        chunk(k + 1, 3, True, True)
        chunk(k + 2, 0, True, True)
        chunk(k + 3, 1, True, True)
        return carry

    lax.fori_loop(0, (ITERS - 6) // 4, loop_body, 0)
    chunk(ITERS - 4, 2, True, True)
    chunk(ITERS - 3, 3, True, True)
    chunk(ITERS - 2, 0, False, False)
    chunk(ITERS - 1, 1, False, False)
    for b in (2, 3, 0, 1):
        pltpu.make_async_copy(bufs[b], acc.at[colidx.at[0]], sss[b]).wait()
    plsc.subcore_barrier()

    base = s * ROWS_PER_SUB
    pltpu.sync_copy(acc.at[pl.ds(base, ROWS_PER_SUB)],
                    g_out.at[c].at[pl.ds(base, ROWS_PER_SUB)])


BN = 1000


def _prep_body(degp_ref, x_ref, dis_ref, p_ref):
    deg = degp_ref[0] + degp_ref[1] + 1.0
    dis = lax.rsqrt(deg)
    dis_ref[...] = dis
    p = dis * x_ref[...]
    p_ref[0] = p[:, :HD]
    p_ref[1] = p[:, HD:]


def _prep(degp2, x):
    return pl.pallas_call(
        _prep_body,
        grid=(N // BN,),
        in_specs=[
            pl.BlockSpec((2, BN, 1), lambda i: (0, i, 0)),
            pl.BlockSpec((BN, D), lambda i: (i, 0)),
        ],
        out_specs=(pl.BlockSpec((BN, 1), lambda i: (i, 0)),
                   pl.BlockSpec((2, BN, HD), lambda i: (0, i, 0))),
        out_shape=(jax.ShapeDtypeStruct((N, 1), jnp.float32),
                   jax.ShapeDtypeStruct((NC, N, HD), jnp.float32)),
    )(degp2, x)


def _tc_body(variant, *refs):
    if variant == "res":
        dis_ref, g_ref, p_ref, w_ref, b_ref, hres_ref, outh_ref, outp_ref = refs
    elif variant == "relu":
        dis_ref, g_ref, p_ref, w_ref, b_ref, outh_ref, outp_ref = refs
    else:
        dis_ref, g_ref, p_ref, w_ref, b_ref, out_ref = refs
    dis = dis_ref[...]
    z0 = dis * (g_ref[0] + p_ref[0])
    z1 = dis * (g_ref[1] + p_ref[1])
    y = (jnp.dot(z0, w_ref[0], preferred_element_type=jnp.float32)
         + jnp.dot(z1, w_ref[1], preferred_element_type=jnp.float32)
         + b_ref[...])
    if variant == "plain":
        out_ref[...] = y
        return
    if variant == "relu":
        hnew = jnp.maximum(y, 0.0)
    else:
        hnew = (hres_ref[...] + jnp.maximum(y, 0.0)) * 0.5
    outh_ref[...] = hnew
    pnew = dis * hnew
    outp_ref[0] = pnew[:, :HD]
    outp_ref[1] = pnew[:, HD:]


def _tc_layer(variant, dis2, g, p, w2, b, hres=None):
    in_specs = [
        pl.BlockSpec((BN, 1), lambda i: (i, 0)),
        pl.BlockSpec((2, BN, HD), lambda i: (0, i, 0)),
        pl.BlockSpec((2, BN, HD), lambda i: (0, i, 0)),
        pl.BlockSpec((2, HD, D), lambda i: (0, 0, 0)),
        pl.BlockSpec((1, D), lambda i: (0, 0)),
    ]
    args = [dis2, g, p, w2, b.reshape(1, D)]
    if variant == "res":
        in_specs.append(pl.BlockSpec((BN, D), lambda i: (i, 0)))
        args.append(hres)
    if variant == "plain":
        out_shape = jax.ShapeDtypeStruct((N, D), jnp.float32)
        out_specs = pl.BlockSpec((BN, D), lambda i: (i, 0))
    else:
        out_shape = (jax.ShapeDtypeStruct((N, D), jnp.float32),
                     jax.ShapeDtypeStruct((NC, N, HD), jnp.float32))
        out_specs = (pl.BlockSpec((BN, D), lambda i: (i, 0)),
                     pl.BlockSpec((2, BN, HD), lambda i: (0, i, 0)))
    return pl.pallas_call(
        functools.partial(_tc_body, variant),
        grid=(N // BN,),
        in_specs=in_specs,
        out_specs=out_specs,
        out_shape=out_shape,
    )(*args)


def kernel(x, edge_index, weights, biases):
    row = edge_index[0]
    col = edge_index[1]

    degp = _deg_kernel(col)
    dis2, p = _prep(degp.reshape(NC, N, 1), x)

    row3 = row.reshape(NS, ITERS, CH)
    col3 = col.reshape(NS, ITERS, CH)

    def agg(p):
        return _agg_kernel(p, row3, col3)

    w2 = [w.reshape(2, HD, D) for w in weights]
    h, p = _tc_layer("relu", dis2, agg(p), p, w2[0], biases[0])
    wi = 1
    for _ in range(3):
        t, pt = _tc_layer("relu", dis2, agg(p), p, w2[wi], biases[wi])
        h, p = _tc_layer("res", dis2, agg(pt), pt, w2[wi + 1],
                         biases[wi + 1], hres=h)
        wi += 2
    out = _tc_layer("plain", dis2, agg(p), p, w2[wi], biases[wi])
    return (out, h)


# speedup vs baseline: 1.0934x; 1.0934x over previous
import functools

import jax
import jax.numpy as jnp
from jax import lax
from jax.experimental import pallas as pl
from jax.experimental.pallas import tpu as pltpu
from jax.experimental.pallas import tpu_sc as plsc

N = 10000
E = 320000
D = 128
NC = 2
NS = 16
NW = NC * NS

CH_DEG = 80
EW_DEG = E // NW
ITERS_DEG = EW_DEG // CH_DEG

CH = 40
EW = E // NW
ITERS = EW // CH

ROWS_PER_SUB = N // NS
STG = 25
NSTG = ROWS_PER_SUB // STG
DEG_CH = 624
DEG_REM = N - NS * DEG_CH

_mesh = plsc.VectorSubcoreMesh(core_axis_name="c", subcore_axis_name="s")


def _fill1d(ref, n, value):
    vec = jnp.full((16,), value, jnp.float32)

    def body(i, carry):
        ref[pl.ds(i * 16, 16)] = vec
        return carry

    lax.fori_loop(0, n // 16, body, 0)


def _fill2d(ref, rows, cols, value):
    vec = jnp.full((16,), value, jnp.float32)

    def rbody(r, carry):
        for cidx in range(cols // 16):
            ref[r, pl.ds(cidx * 16, 16)] = vec
        return carry

    lax.fori_loop(0, rows, rbody, 0)


@functools.partial(
    pl.kernel,
    out_type=jax.ShapeDtypeStruct((NC * N,), jnp.float32),
    mesh=_mesh,
    compiler_params=pltpu.CompilerParams(use_tc_tiling_on_sc=False),
    scratch_types=[
        pltpu.VMEM_SHARED((N,), jnp.float32),
        pltpu.VMEM((CH_DEG,), jnp.int32),
        pltpu.VMEM((CH_DEG,), jnp.float32),
        pltpu.VMEM((DEG_CH,), jnp.float32),
    ],
)
def _deg_kernel(col_hbm, deg_out, acc, colbuf, onesbuf, stage):
    c = lax.axis_index("c")
    s = lax.axis_index("s")
    wid = c * NS + s

    _fill1d(stage, DEG_CH, 0.0)
    _fill1d(onesbuf, CH_DEG, 1.0)
    pltpu.sync_copy(stage, acc.at[pl.ds(s * DEG_CH, DEG_CH)])

    @pl.when(s == NS - 1)
    def _zero_tail():
        pltpu.sync_copy(stage.at[pl.ds(0, DEG_REM)],
                        acc.at[pl.ds(NS * DEG_CH, DEG_REM)])

    plsc.subcore_barrier()

    def body(i, carry):
        base = wid * EW_DEG + i * CH_DEG
        pltpu.sync_copy(col_hbm.at[pl.ds(base, CH_DEG)], colbuf)
        pltpu.sync_copy(onesbuf, acc.at[colbuf], add=True)
        return carry

    lax.fori_loop(0, ITERS_DEG, body, 0)
    plsc.subcore_barrier()

    pltpu.sync_copy(acc.at[pl.ds(s * DEG_CH, DEG_CH)], stage)
    pltpu.sync_copy(stage, deg_out.at[pl.ds(c * N + s * DEG_CH, DEG_CH)])

    @pl.when(s == NS - 1)
    def _copy_tail():
        pltpu.sync_copy(acc.at[pl.ds(NS * DEG_CH, DEG_REM)],
                        stage.at[pl.ds(0, DEG_REM)])
        pltpu.sync_copy(stage.at[pl.ds(0, DEG_REM)],
                        deg_out.at[pl.ds(c * N + NS * DEG_CH, DEG_REM)])


@functools.partial(
    pl.kernel,
    out_type=jax.ShapeDtypeStruct((NC, N, D), jnp.float32),
    mesh=_mesh,
    compiler_params=pltpu.CompilerParams(use_tc_tiling_on_sc=False),
    scratch_types=[
        pltpu.VMEM_SHARED((N, D), jnp.float32),
        pltpu.VMEM((ITERS, CH), jnp.int32),
        pltpu.VMEM((ITERS, CH), jnp.int32),
        pltpu.VMEM((CH, D), jnp.float32),
        pltpu.VMEM((CH, D), jnp.float32),
        pltpu.VMEM((CH, D), jnp.float32),
        pltpu.VMEM((CH, D), jnp.float32),
        pltpu.VMEM((STG, D), jnp.float32),
        pltpu.SemaphoreType.DMA,
        pltpu.SemaphoreType.DMA,
        pltpu.SemaphoreType.DMA,
        pltpu.SemaphoreType.DMA,
        pltpu.SemaphoreType.DMA,
        pltpu.SemaphoreType.DMA,
        pltpu.SemaphoreType.DMA,
        pltpu.SemaphoreType.DMA,
        pltpu.SemaphoreType.DMA,
    ],
)
def _agg_kernel(p_hbm, row_hbm, col_hbm, g_out,
                acc, rowidx, colidx, buf0, buf1, buf2, buf3, stage,
                sg0, sg1, sg2, sg3, ss0, ss1, ss2, ss3, sidx):
    c = lax.axis_index("c")
    s = lax.axis_index("s")
    wid = c * NS + s
    bufs = (buf0, buf1, buf2, buf3)
    sgs = (sg0, sg1, sg2, sg3)
    sss = (ss0, ss1, ss2, ss3)
    phalf = p_hbm

    di_r = pltpu.async_copy(row_hbm.at[wid], rowidx, sidx)
    di_c = pltpu.async_copy(col_hbm.at[wid], colidx, sidx)

    _fill2d(stage, STG, D, 0.0)
    di_r.wait()
    di_c.wait()

    pltpu.async_copy(phalf.at[rowidx.at[0]], bufs[0], sgs[0])
    pltpu.async_copy(phalf.at[rowidx.at[1]], bufs[1], sgs[1])

    def zbody(k, carry):
        pltpu.sync_copy(stage, acc.at[pl.ds(s * ROWS_PER_SUB + k * STG, STG)])
        return carry

    lax.fori_loop(0, NSTG, zbody, 0)
    plsc.subcore_barrier()

    def chunk(k, b, wait_prev_scatter, start_next_gather):
        buf, sg, ss = bufs[b], sgs[b], sss[b]
        pltpu.make_async_copy(phalf.at[rowidx.at[k]], buf, sg).wait()
        pltpu.async_copy(buf, acc.at[colidx.at[k]], ss, add=True)
        if start_next_gather:
            b2 = (b + 2) % 4
            if wait_prev_scatter:
                pltpu.make_async_copy(bufs[b2], acc.at[colidx.at[k]],
                                      sss[b2]).wait()
            pltpu.async_copy(phalf.at[rowidx.at[k + 2]], bufs[b2], sgs[b2])

    chunk(0, 0, False, True)
    chunk(1, 1, False, True)

    def loop_body(k4, carry):
        k = 2 + 4 * k4
        chunk(k, 2, True, True)
        chunk(k + 1, 3, True, True)
        chunk(k + 2, 0, True, True)
        chunk(k + 3, 1, True, True)
        return carry

    lax.fori_loop(0, (ITERS - 6) // 4, loop_body, 0)
    chunk(ITERS - 4, 2, True, True)
    chunk(ITERS - 3, 3, True, True)
    chunk(ITERS - 2, 0, False, False)
    chunk(ITERS - 1, 1, False, False)
    for b in (2, 3, 0, 1):
        pltpu.make_async_copy(bufs[b], acc.at[colidx.at[0]], sss[b]).wait()
    plsc.subcore_barrier()

    base = s * ROWS_PER_SUB
    pltpu.sync_copy(acc.at[pl.ds(base, ROWS_PER_SUB)],
                    g_out.at[c].at[pl.ds(base, ROWS_PER_SUB)])


BN = 1000


def _prep_body(degp_ref, x_ref, dis_ref, p_ref):
    deg = degp_ref[0] + degp_ref[1] + 1.0
    dis = lax.rsqrt(deg)
    dis_ref[...] = dis
    p_ref[...] = dis * x_ref[...]


def _prep(degp2, x):
    return pl.pallas_call(
        _prep_body,
        grid=(N // BN,),
        in_specs=[
            pl.BlockSpec((2, BN, 1), lambda i: (0, i, 0)),
            pl.BlockSpec((BN, D), lambda i: (i, 0)),
        ],
        out_specs=(pl.BlockSpec((BN, 1), lambda i: (i, 0)),
                   pl.BlockSpec((BN, D), lambda i: (i, 0))),
        out_shape=(jax.ShapeDtypeStruct((N, 1), jnp.float32),
                   jax.ShapeDtypeStruct((N, D), jnp.float32)),
    )(degp2, x)


def _tc_body(variant, *refs):
    if variant == "res":
        dis_ref, g_ref, p_ref, w_ref, b_ref, hres_ref, outh_ref, outp_ref = refs
    elif variant == "relu":
        dis_ref, g_ref, p_ref, w_ref, b_ref, outh_ref, outp_ref = refs
    else:
        dis_ref, g_ref, p_ref, w_ref, b_ref, out_ref = refs
    dis = dis_ref[...]
    z = dis * (g_ref[0] + g_ref[1] + p_ref[...])
    y = jnp.dot(z, w_ref[...], preferred_element_type=jnp.float32) + b_ref[...]
    if variant == "plain":
        out_ref[...] = y
        return
    if variant == "relu":
        hnew = jnp.maximum(y, 0.0)
    else:
        hnew = (hres_ref[...] + jnp.maximum(y, 0.0)) * 0.5
    outh_ref[...] = hnew
    outp_ref[...] = dis * hnew


def _tc_layer(variant, dis2, g, p, w, b, hres=None):
    in_specs = [
        pl.BlockSpec((BN, 1), lambda i: (i, 0)),
        pl.BlockSpec((2, BN, D), lambda i: (0, i, 0)),
        pl.BlockSpec((BN, D), lambda i: (i, 0)),
        pl.BlockSpec((D, D), lambda i: (0, 0)),
        pl.BlockSpec((1, D), lambda i: (0, 0)),
    ]
    args = [dis2, g, p, w, b.reshape(1, D)]
    if variant == "res":
        in_specs.append(pl.BlockSpec((BN, D), lambda i: (i, 0)))
        args.append(hres)
    if variant == "plain":
        out_shape = jax.ShapeDtypeStruct((N, D), jnp.float32)
        out_specs = pl.BlockSpec((BN, D), lambda i: (i, 0))
    else:
        out_shape = (jax.ShapeDtypeStruct((N, D), jnp.float32),
                     jax.ShapeDtypeStruct((N, D), jnp.float32))
        out_specs = (pl.BlockSpec((BN, D), lambda i: (i, 0)),
                     pl.BlockSpec((BN, D), lambda i: (i, 0)))
    return pl.pallas_call(
        functools.partial(_tc_body, variant),
        grid=(N // BN,),
        in_specs=in_specs,
        out_specs=out_specs,
        out_shape=out_shape,
    )(*args)


def kernel(x, edge_index, weights, biases):
    row = edge_index[0]
    col = edge_index[1]

    degp = _deg_kernel(col)
    dis2, p = _prep(degp.reshape(NC, N, 1), x)

    row3 = row.reshape(NW, ITERS, CH)
    col3 = col.reshape(NW, ITERS, CH)

    def agg(p):
        return _agg_kernel(p, row3, col3)

    h, p = _tc_layer("relu", dis2, agg(p), p, weights[0], biases[0])
    wi = 1
    for _ in range(3):
        t, pt = _tc_layer("relu", dis2, agg(p), p, weights[wi], biases[wi])
        h, p = _tc_layer("res", dis2, agg(pt), pt, weights[wi + 1],
                         biases[wi + 1], hres=h)
        wi += 2
    out = _tc_layer("plain", dis2, agg(p), p, weights[wi], biases[wi])
    return (out, h)


# speedup vs baseline: 1.3209x; 1.2081x over previous
import functools

import jax
import jax.numpy as jnp
from jax import lax
from jax.experimental import pallas as pl
from jax.experimental.pallas import tpu as pltpu
from jax.experimental.pallas import tpu_sc as plsc

N = 10000
E = 320000
D = 128
NC = 2
NS = 16
NW = NC * NS

CH_DEG = 80
EW_DEG = E // NW
ITERS_DEG = EW_DEG // CH_DEG

CH = 80
EW = E // NW
ITERS = EW // CH

ROWS_PER_SUB = N // NS
STG = 25
NSTG = ROWS_PER_SUB // STG
DEG_CH = 624
DEG_REM = N - NS * DEG_CH

_mesh = plsc.VectorSubcoreMesh(core_axis_name="c", subcore_axis_name="s")


def _fill1d(ref, n, value):
    vec = jnp.full((16,), value, jnp.float32)

    def body(i, carry):
        ref[pl.ds(i * 16, 16)] = vec
        return carry

    lax.fori_loop(0, n // 16, body, 0)


def _fill2d(ref, rows, cols, value):
    vec = jnp.full((16,), value, jnp.float32)

    def rbody(r, carry):
        for cidx in range(cols // 16):
            ref[r, pl.ds(cidx * 16, 16)] = vec
        return carry

    lax.fori_loop(0, rows, rbody, 0)


@functools.partial(
    pl.kernel,
    out_type=jax.ShapeDtypeStruct((NC * N,), jnp.float32),
    mesh=_mesh,
    compiler_params=pltpu.CompilerParams(use_tc_tiling_on_sc=False),
    scratch_types=[
        pltpu.VMEM_SHARED((N,), jnp.float32),
        pltpu.VMEM((ITERS_DEG, CH_DEG), jnp.int32),
        pltpu.VMEM((CH_DEG,), jnp.float32),
        pltpu.VMEM((DEG_CH,), jnp.float32),
        pltpu.SemaphoreType.DMA,
        pltpu.SemaphoreType.DMA,
        pltpu.SemaphoreType.DMA,
        pltpu.SemaphoreType.DMA,
        pltpu.SemaphoreType.DMA,
    ],
)
def _deg_kernel(col_hbm, deg_out, acc, colidx, onesbuf, stage,
                d0, d1, d2, d3, sidx):
    c = lax.axis_index("c")
    s = lax.axis_index("s")
    sems = (d0, d1, d2, d3)

    di = pltpu.async_copy(col_hbm.at[c * NS + s], colidx, sidx)
    _fill1d(stage, DEG_CH, 0.0)
    _fill1d(onesbuf, CH_DEG, 1.0)
    pltpu.sync_copy(stage, acc.at[pl.ds(s * DEG_CH, DEG_CH)])

    @pl.when(s == NS - 1)
    def _zero_tail():
        pltpu.sync_copy(stage.at[pl.ds(0, DEG_REM)],
                        acc.at[pl.ds(NS * DEG_CH, DEG_REM)])

    di.wait()
    plsc.subcore_barrier()

    def body(i, carry):
        k = 4 * i
        for j in range(4):
            pltpu.async_copy(onesbuf, acc.at[colidx.at[k + j]], sems[j],
                             add=True)
        for j in range(4):
            pltpu.make_async_copy(onesbuf, acc.at[colidx.at[k + j]],
                                  sems[j]).wait()
        return carry

    lax.fori_loop(0, ITERS_DEG // 4, body, 0)
    pltpu.sync_copy(onesbuf, acc.at[colidx.at[ITERS_DEG - 1]], add=True)
    plsc.subcore_barrier()

    pltpu.sync_copy(acc.at[pl.ds(s * DEG_CH, DEG_CH)], stage)
    pltpu.sync_copy(stage, deg_out.at[pl.ds(c * N + s * DEG_CH, DEG_CH)])

    @pl.when(s == NS - 1)
    def _copy_tail():
        pltpu.sync_copy(acc.at[pl.ds(NS * DEG_CH, DEG_REM)],
                        stage.at[pl.ds(0, DEG_REM)])
        pltpu.sync_copy(stage.at[pl.ds(0, DEG_REM)],
                        deg_out.at[pl.ds(c * N + NS * DEG_CH, DEG_REM)])


@functools.partial(
    pl.kernel,
    out_type=jax.ShapeDtypeStruct((NC, N, D), jnp.float32),
    mesh=_mesh,
    compiler_params=pltpu.CompilerParams(use_tc_tiling_on_sc=False),
    scratch_types=[
        pltpu.VMEM_SHARED((N, D), jnp.float32),
        pltpu.VMEM((ITERS, CH), jnp.int32),
        pltpu.VMEM((ITERS, CH), jnp.int32),
        pltpu.VMEM((CH, D), jnp.float32),
        pltpu.VMEM((CH, D), jnp.float32),
        pltpu.VMEM((STG, D), jnp.float32),
        pltpu.SemaphoreType.DMA,
        pltpu.SemaphoreType.DMA,
        pltpu.SemaphoreType.DMA,
        pltpu.SemaphoreType.DMA,
        pltpu.SemaphoreType.DMA,
    ],
)
def _agg_kernel(p_hbm, row_hbm, col_hbm, g_out,
                acc, rowidx, colidx, buf0, buf1, stage,
                sg0, sg1, ss0, ss1, sidx):
    c = lax.axis_index("c")
    s = lax.axis_index("s")
    wid = c * NS + s
    bufs = (buf0, buf1)
    sgs = (sg0, sg1)
    sss = (ss0, ss1)

    di_r = pltpu.async_copy(row_hbm.at[wid], rowidx, sidx)
    di_c = pltpu.async_copy(col_hbm.at[wid], colidx, sidx)

    _fill2d(stage, STG, D, 0.0)
    di_r.wait()
    di_c.wait()

    pltpu.async_copy(p_hbm.at[rowidx.at[0]], bufs[0], sgs[0])
    pltpu.async_copy(p_hbm.at[rowidx.at[1]], bufs[1], sgs[1])

    def zbody(k, carry):
        pltpu.sync_copy(stage, acc.at[pl.ds(s * ROWS_PER_SUB + k * STG, STG)])
        return carry

    lax.fori_loop(0, NSTG, zbody, 0)
    plsc.subcore_barrier()

    def chunk(k, b):
        buf, sg, ss = bufs[b], sgs[b], sss[b]
        pltpu.make_async_copy(p_hbm.at[rowidx.at[k]], buf, sg).wait()
        pltpu.async_copy(buf, acc.at[colidx.at[k]], ss, add=True)
        pltpu.make_async_copy(buf, acc.at[colidx.at[k]], ss).wait()

        @pl.when(k + 2 < ITERS)
        def _next_gather():
            pltpu.async_copy(p_hbm.at[rowidx.at[k + 2]], buf, sg)

    def loop_body(k2, carry):
        chunk(2 * k2, 0)
        chunk(2 * k2 + 1, 1)
        return carry

    lax.fori_loop(0, ITERS // 2, loop_body, 0)
    chunk(ITERS - 1, 0)
    plsc.subcore_barrier()

    base = s * ROWS_PER_SUB
    pltpu.sync_copy(acc.at[pl.ds(base, ROWS_PER_SUB)],
                    g_out.at[c].at[pl.ds(base, ROWS_PER_SUB)])


BN = 1000


def _prep_body(degp_ref, x_ref, dis_ref, p_ref):
    deg = degp_ref[0] + degp_ref[1] + 1.0
    dis = lax.rsqrt(deg)
    dis_ref[...] = dis
    p_ref[...] = dis * x_ref[...]


def _prep(degp2, x):
    return pl.pallas_call(
        _prep_body,
        grid=(N // BN,),
        in_specs=[
            pl.BlockSpec((2, BN, 1), lambda i: (0, i, 0)),
            pl.BlockSpec((BN, D), lambda i: (i, 0)),
        ],
        out_specs=(pl.BlockSpec((BN, 1), lambda i: (i, 0)),
                   pl.BlockSpec((BN, D), lambda i: (i, 0))),
        out_shape=(jax.ShapeDtypeStruct((N, 1), jnp.float32),
                   jax.ShapeDtypeStruct((N, D), jnp.float32)),
    )(degp2, x)


def _tc_body(variant, *refs):
    if variant == "res":
        dis_ref, g_ref, p_ref, w_ref, b_ref, hres_ref, outh_ref, outp_ref = refs
    elif variant == "relu":
        dis_ref, g_ref, p_ref, w_ref, b_ref, outh_ref, outp_ref = refs
    elif variant == "mid":
        dis_ref, g_ref, p_ref, w_ref, b_ref, outp_ref = refs
    else:
        dis_ref, g_ref, p_ref, w_ref, b_ref, out_ref = refs
    dis = dis_ref[...]
    z = dis * (g_ref[0] + g_ref[1] + p_ref[...])
    y = jnp.dot(z, w_ref[...], preferred_element_type=jnp.float32) + b_ref[...]
    if variant == "plain":
        out_ref[...] = y
        return
    if variant == "mid":
        outp_ref[...] = dis * jnp.maximum(y, 0.0)
        return
    if variant == "relu":
        hnew = jnp.maximum(y, 0.0)
    else:
        hnew = (hres_ref[...] + jnp.maximum(y, 0.0)) * 0.5
    outh_ref[...] = hnew
    outp_ref[...] = dis * hnew


def _tc_layer(variant, dis2, g, p, w, b, hres=None):
    in_specs = [
        pl.BlockSpec((BN, 1), lambda i: (i, 0)),
        pl.BlockSpec((2, BN, D), lambda i: (0, i, 0)),
        pl.BlockSpec((BN, D), lambda i: (i, 0)),
        pl.BlockSpec((D, D), lambda i: (0, 0)),
        pl.BlockSpec((1, D), lambda i: (0, 0)),
    ]
    args = [dis2, g, p, w, b.reshape(1, D)]
    if variant == "res":
        in_specs.append(pl.BlockSpec((BN, D), lambda i: (i, 0)))
        args.append(hres)
    if variant in ("plain", "mid"):
        out_shape = jax.ShapeDtypeStruct((N, D), jnp.float32)
        out_specs = pl.BlockSpec((BN, D), lambda i: (i, 0))
    else:
        out_shape = (jax.ShapeDtypeStruct((N, D), jnp.float32),
                     jax.ShapeDtypeStruct((N, D), jnp.float32))
        out_specs = (pl.BlockSpec((BN, D), lambda i: (i, 0)),
                     pl.BlockSpec((BN, D), lambda i: (i, 0)))
    return pl.pallas_call(
        functools.partial(_tc_body, variant),
        grid=(N // BN,),
        in_specs=in_specs,
        out_specs=out_specs,
        out_shape=out_shape,
    )(*args)


def kernel(x, edge_index, weights, biases):
    row = edge_index[0]
    col = edge_index[1]
    row3 = row.reshape(NW, ITERS, CH)
    col3 = col.reshape(NW, ITERS, CH)

    degp = _deg_kernel(col3)
    dis2, p = _prep(degp.reshape(NC, N, 1), x)

    def agg(p):
        return _agg_kernel(p, row3, col3)

    h, p = _tc_layer("relu", dis2, agg(p), p, weights[0], biases[0])
    wi = 1
    for _ in range(3):
        pt = _tc_layer("mid", dis2, agg(p), p, weights[wi], biases[wi])
        h, p = _tc_layer("res", dis2, agg(pt), pt, weights[wi + 1],
                         biases[wi + 1], hres=h)
        wi += 2
    out = _tc_layer("plain", dis2, agg(p), p, weights[wi], biases[wi])
    return (out, h)


# speedup vs baseline: 1.3457x; 1.0187x over previous
import functools

import jax
import jax.numpy as jnp
from jax import lax
from jax.experimental import pallas as pl
from jax.experimental.pallas import tpu as pltpu
from jax.experimental.pallas import tpu_sc as plsc

N = 10000
E = 320000
D = 128
NC = 2
NS = 16
NW = NC * NS

CH_DEG = 80
EW_DEG = E // NW
ITERS_DEG = EW_DEG // CH_DEG

CH = 80
EW = E // NW
ITERS = EW // CH

ROWS_PER_SUB = N // NS
STG = 25
NSTG = ROWS_PER_SUB // STG
DEG_CH = 624
DEG_REM = N - NS * DEG_CH

_mesh = plsc.VectorSubcoreMesh(core_axis_name="c", subcore_axis_name="s")


def _fill1d(ref, n, value):
    vec = jnp.full((16,), value, jnp.float32)

    def body(i, carry):
        ref[pl.ds(i * 16, 16)] = vec
        return carry

    lax.fori_loop(0, n // 16, body, 0)


def _fill2d(ref, rows, cols, value):
    vec = jnp.full((16,), value, jnp.float32)

    def rbody(r, carry):
        for cidx in range(cols // 16):
            ref[r, pl.ds(cidx * 16, 16)] = vec
        return carry

    lax.fori_loop(0, rows, rbody, 0)


@functools.partial(
    pl.kernel,
    out_type=jax.ShapeDtypeStruct((NC * N,), jnp.float32),
    mesh=_mesh,
    compiler_params=pltpu.CompilerParams(use_tc_tiling_on_sc=False),
    scratch_types=[
        pltpu.VMEM_SHARED((N,), jnp.float32),
        pltpu.VMEM((ITERS_DEG, CH_DEG), jnp.int32),
        pltpu.VMEM((CH_DEG,), jnp.float32),
        pltpu.VMEM((DEG_CH,), jnp.float32),
        pltpu.SemaphoreType.DMA,
        pltpu.SemaphoreType.DMA,
        pltpu.SemaphoreType.DMA,
        pltpu.SemaphoreType.DMA,
        pltpu.SemaphoreType.DMA,
    ],
)
def _deg_kernel(col_hbm, deg_out, acc, colidx, onesbuf, stage,
                d0, d1, d2, d3, sidx):
    c = lax.axis_index("c")
    s = lax.axis_index("s")
    sems = (d0, d1, d2, d3)

    di = pltpu.async_copy(col_hbm.at[c * NS + s], colidx, sidx)
    _fill1d(stage, DEG_CH, 0.0)
    _fill1d(onesbuf, CH_DEG, 1.0)
    pltpu.sync_copy(stage, acc.at[pl.ds(s * DEG_CH, DEG_CH)])

    @pl.when(s == NS - 1)
    def _zero_tail():
        pltpu.sync_copy(stage.at[pl.ds(0, DEG_REM)],
                        acc.at[pl.ds(NS * DEG_CH, DEG_REM)])

    di.wait()
    plsc.subcore_barrier()

    def body(i, carry):
        k = 4 * i
        for j in range(4):
            pltpu.async_copy(onesbuf, acc.at[colidx.at[k + j]], sems[j],
                             add=True)
        for j in range(4):
            pltpu.make_async_copy(onesbuf, acc.at[colidx.at[k + j]],
                                  sems[j]).wait()
        return carry

    lax.fori_loop(0, ITERS_DEG // 4, body, 0)
    pltpu.sync_copy(onesbuf, acc.at[colidx.at[ITERS_DEG - 1]], add=True)
    plsc.subcore_barrier()

    pltpu.sync_copy(acc.at[pl.ds(s * DEG_CH, DEG_CH)], stage)
    pltpu.sync_copy(stage, deg_out.at[pl.ds(c * N + s * DEG_CH, DEG_CH)])

    @pl.when(s == NS - 1)
    def _copy_tail():
        pltpu.sync_copy(acc.at[pl.ds(NS * DEG_CH, DEG_REM)],
                        stage.at[pl.ds(0, DEG_REM)])
        pltpu.sync_copy(stage.at[pl.ds(0, DEG_REM)],
                        deg_out.at[pl.ds(c * N + NS * DEG_CH, DEG_REM)])


@functools.partial(
    pl.kernel,
    out_type=jax.ShapeDtypeStruct((NC, N, D), jnp.float32),
    mesh=_mesh,
    compiler_params=pltpu.CompilerParams(use_tc_tiling_on_sc=False),
    scratch_types=[
        pltpu.VMEM_SHARED((N, D), jnp.float32),
        pltpu.VMEM((ITERS, CH), jnp.int32),
        pltpu.VMEM((ITERS, CH), jnp.int32),
        pltpu.VMEM((CH, D), jnp.float32),
        pltpu.VMEM((CH, D), jnp.float32),
        pltpu.VMEM((STG, D), jnp.float32),
        pltpu.SemaphoreType.DMA,
        pltpu.SemaphoreType.DMA,
        pltpu.SemaphoreType.DMA,
        pltpu.SemaphoreType.DMA,
        pltpu.SemaphoreType.DMA,
    ],
)
def _agg_kernel(p_hbm, row_hbm, col_hbm, g_out,
                acc, rowidx, colidx, buf0, buf1, stage,
                sg0, sg1, ss0, ss1, sidx):
    c = lax.axis_index("c")
    s = lax.axis_index("s")
    wid = c * NS + s
    bufs = (buf0, buf1)
    sgs = (sg0, sg1)
    sss = (ss0, ss1)

    di_r = pltpu.async_copy(row_hbm.at[wid], rowidx, sidx)
    di_c = pltpu.async_copy(col_hbm.at[wid], colidx, sidx)

    _fill2d(stage, STG, D, 0.0)
    di_r.wait()
    di_c.wait()

    pltpu.async_copy(p_hbm.at[rowidx.at[0]], bufs[0], sgs[0])
    pltpu.async_copy(p_hbm.at[rowidx.at[1]], bufs[1], sgs[1])

    def zbody(k, carry):
        pltpu.sync_copy(stage, acc.at[pl.ds(s * ROWS_PER_SUB + k * STG, STG)])
        return carry

    lax.fori_loop(0, NSTG, zbody, 0)
    plsc.subcore_barrier()

    def chunk(k, b):
        buf, sg, ss = bufs[b], sgs[b], sss[b]
        pltpu.make_async_copy(p_hbm.at[rowidx.at[k]], buf, sg).wait()
        pltpu.async_copy(buf, acc.at[colidx.at[k]], ss, add=True)
        pltpu.make_async_copy(buf, acc.at[colidx.at[k]], ss).wait()

        @pl.when(k + 2 < ITERS)
        def _next_gather():
            pltpu.async_copy(p_hbm.at[rowidx.at[k + 2]], buf, sg)

    def loop_body(k2, carry):
        chunk(2 * k2, 0)
        chunk(2 * k2 + 1, 1)
        return carry

    lax.fori_loop(0, ITERS // 2, loop_body, 0)
    chunk(ITERS - 1, 0)
    plsc.subcore_barrier()

    base = s * ROWS_PER_SUB
    pltpu.sync_copy(acc.at[pl.ds(base, ROWS_PER_SUB)],
                    g_out.at[c].at[pl.ds(base, ROWS_PER_SUB)])


BN = 2000


def _prep_body(degp_ref, x_ref, dis_ref, p_ref):
    deg = degp_ref[0] + degp_ref[1] + 1.0
    dis = lax.rsqrt(deg)
    dis_ref[...] = dis
    p_ref[...] = dis * x_ref[...]


def _prep(degp2, x):
    return pl.pallas_call(
        _prep_body,
        grid=(N // BN,),
        in_specs=[
            pl.BlockSpec((2, BN, 1), lambda i: (0, i, 0)),
            pl.BlockSpec((BN, D), lambda i: (i, 0)),
        ],
        out_specs=(pl.BlockSpec((BN, 1), lambda i: (i, 0)),
                   pl.BlockSpec((BN, D), lambda i: (i, 0))),
        out_shape=(jax.ShapeDtypeStruct((N, 1), jnp.float32),
                   jax.ShapeDtypeStruct((N, D), jnp.float32)),
    )(degp2, x)


def _tc_body(variant, *refs):
    if variant == "res":
        dis_ref, g_ref, p_ref, w_ref, b_ref, hres_ref, outh_ref, outp_ref = refs
    elif variant == "relu":
        dis_ref, g_ref, p_ref, w_ref, b_ref, outh_ref, outp_ref = refs
    elif variant == "mid":
        dis_ref, g_ref, p_ref, w_ref, b_ref, outp_ref = refs
    else:
        dis_ref, g_ref, p_ref, w_ref, b_ref, out_ref = refs
    dis = dis_ref[...]
    z = dis * (g_ref[0] + g_ref[1] + p_ref[...])
    y = jnp.dot(z, w_ref[...], preferred_element_type=jnp.float32) + b_ref[...]
    if variant == "plain":
        out_ref[...] = y
        return
    if variant == "mid":
        outp_ref[...] = dis * jnp.maximum(y, 0.0)
        return
    if variant == "relu":
        hnew = jnp.maximum(y, 0.0)
    else:
        hnew = (hres_ref[...] + jnp.maximum(y, 0.0)) * 0.5
    outh_ref[...] = hnew
    outp_ref[...] = dis * hnew


def _tc_layer(variant, dis2, g, p, w, b, hres=None):
    in_specs = [
        pl.BlockSpec((BN, 1), lambda i: (i, 0)),
        pl.BlockSpec((2, BN, D), lambda i: (0, i, 0)),
        pl.BlockSpec((BN, D), lambda i: (i, 0)),
        pl.BlockSpec((D, D), lambda i: (0, 0)),
        pl.BlockSpec((1, D), lambda i: (0, 0)),
    ]
    args = [dis2, g, p, w, b.reshape(1, D)]
    if variant == "res":
        in_specs.append(pl.BlockSpec((BN, D), lambda i: (i, 0)))
        args.append(hres)
    if variant in ("plain", "mid"):
        out_shape = jax.ShapeDtypeStruct((N, D), jnp.float32)
        out_specs = pl.BlockSpec((BN, D), lambda i: (i, 0))
    else:
        out_shape = (jax.ShapeDtypeStruct((N, D), jnp.float32),
                     jax.ShapeDtypeStruct((N, D), jnp.float32))
        out_specs = (pl.BlockSpec((BN, D), lambda i: (i, 0)),
                     pl.BlockSpec((BN, D), lambda i: (i, 0)))
    return pl.pallas_call(
        functools.partial(_tc_body, variant),
        grid=(N // BN,),
        in_specs=in_specs,
        out_specs=out_specs,
        out_shape=out_shape,
    )(*args)


def kernel(x, edge_index, weights, biases):
    row = edge_index[0]
    col = edge_index[1]
    row3 = row.reshape(NW, ITERS, CH)
    col3 = col.reshape(NW, ITERS, CH)

    degp = _deg_kernel(col3)
    dis2, p = _prep(degp.reshape(NC, N, 1), x)

    def agg(p):
        return _agg_kernel(p, row3, col3)

    h, p = _tc_layer("relu", dis2, agg(p), p, weights[0], biases[0])
    wi = 1
    for _ in range(3):
        pt = _tc_layer("mid", dis2, agg(p), p, weights[wi], biases[wi])
        h, p = _tc_layer("res", dis2, agg(pt), pt, weights[wi + 1],
                         biases[wi + 1], hres=h)
        wi += 2
    out = _tc_layer("plain", dis2, agg(p), p, weights[wi], biases[wi])
    return (out, h)


# speedup vs baseline: 1.3509x; 1.0038x over previous
import functools

import jax
import jax.numpy as jnp
from jax import lax
from jax.experimental import pallas as pl
from jax.experimental.pallas import tpu as pltpu
from jax.experimental.pallas import tpu_sc as plsc

N = 10000
E = 320000
D = 128
NC = 2
NS = 16
NW = NC * NS

CH_DEG = 80
EW_DEG = E // NW
ITERS_DEG = EW_DEG // CH_DEG

CH = 80
EW = E // NW
ITERS = EW // CH

ROWS_PER_SUB = N // NS
STG = 25
NSTG = ROWS_PER_SUB // STG
DEG_CH = 624
DEG_REM = N - NS * DEG_CH

_mesh = plsc.VectorSubcoreMesh(core_axis_name="c", subcore_axis_name="s")


def _fill1d(ref, n, value):
    vec = jnp.full((16,), value, jnp.float32)

    def body(i, carry):
        ref[pl.ds(i * 16, 16)] = vec
        return carry

    lax.fori_loop(0, n // 16, body, 0)


def _fill2d(ref, rows, cols, value):
    vec = jnp.full((16,), value, jnp.float32)

    def rbody(r, carry):
        for cidx in range(cols // 16):
            ref[r, pl.ds(cidx * 16, 16)] = vec
        return carry

    lax.fori_loop(0, rows, rbody, 0)


@functools.partial(
    pl.kernel,
    out_type=jax.ShapeDtypeStruct((NC * N,), jnp.float32),
    mesh=_mesh,
    compiler_params=pltpu.CompilerParams(use_tc_tiling_on_sc=False),
    scratch_types=[
        pltpu.VMEM_SHARED((N,), jnp.float32),
        pltpu.VMEM((ITERS_DEG, CH_DEG), jnp.int32),
        pltpu.VMEM((CH_DEG,), jnp.float32),
        pltpu.VMEM((DEG_CH,), jnp.float32),
        pltpu.SemaphoreType.DMA,
        pltpu.SemaphoreType.DMA,
        pltpu.SemaphoreType.DMA,
        pltpu.SemaphoreType.DMA,
        pltpu.SemaphoreType.DMA,
    ],
)
def _deg_kernel(col_hbm, deg_out, acc, colidx, onesbuf, stage,
                d0, d1, d2, d3, sidx):
    c = lax.axis_index("c")
    s = lax.axis_index("s")
    sems = (d0, d1, d2, d3)

    di = pltpu.async_copy(col_hbm.at[c * NS + s], colidx, sidx)
    _fill1d(stage, DEG_CH, 0.0)
    _fill1d(onesbuf, CH_DEG, 1.0)
    pltpu.sync_copy(stage, acc.at[pl.ds(s * DEG_CH, DEG_CH)])

    @pl.when(s == NS - 1)
    def _zero_tail():
        pltpu.sync_copy(stage.at[pl.ds(0, DEG_REM)],
                        acc.at[pl.ds(NS * DEG_CH, DEG_REM)])

    di.wait()
    plsc.subcore_barrier()

    def body(i, carry):
        k = 4 * i
        for j in range(4):
            pltpu.async_copy(onesbuf, acc.at[colidx.at[k + j]], sems[j],
                             add=True)
        for j in range(4):
            pltpu.make_async_copy(onesbuf, acc.at[colidx.at[k + j]],
                                  sems[j]).wait()
        return carry

    lax.fori_loop(0, ITERS_DEG // 4, body, 0)
    pltpu.sync_copy(onesbuf, acc.at[colidx.at[ITERS_DEG - 1]], add=True)
    plsc.subcore_barrier()

    pltpu.sync_copy(acc.at[pl.ds(s * DEG_CH, DEG_CH)], stage)
    pltpu.sync_copy(stage, deg_out.at[pl.ds(c * N + s * DEG_CH, DEG_CH)])

    @pl.when(s == NS - 1)
    def _copy_tail():
        pltpu.sync_copy(acc.at[pl.ds(NS * DEG_CH, DEG_REM)],
                        stage.at[pl.ds(0, DEG_REM)])
        pltpu.sync_copy(stage.at[pl.ds(0, DEG_REM)],
                        deg_out.at[pl.ds(c * N + NS * DEG_CH, DEG_REM)])


@functools.partial(
    pl.kernel,
    out_type=jax.ShapeDtypeStruct((NC, N, D), jnp.float32),
    mesh=_mesh,
    compiler_params=pltpu.CompilerParams(use_tc_tiling_on_sc=False),
    scratch_types=[
        pltpu.VMEM_SHARED((N, D), jnp.float32),
        pltpu.VMEM((ITERS, CH), jnp.int32),
        pltpu.VMEM((ITERS, CH), jnp.int32),
        pltpu.VMEM((CH, D), jnp.float32),
        pltpu.VMEM((CH, D), jnp.float32),
        pltpu.VMEM((STG, D), jnp.float32),
        pltpu.SemaphoreType.DMA,
        pltpu.SemaphoreType.DMA,
        pltpu.SemaphoreType.DMA,
        pltpu.SemaphoreType.DMA,
        pltpu.SemaphoreType.DMA,
        pltpu.SemaphoreType.DMA,
        pltpu.SemaphoreType.DMA,
        pltpu.SemaphoreType.DMA,
        pltpu.SemaphoreType.DMA,
    ],
)
def _agg_kernel(p_hbm, row_hbm, col_hbm, g_out,
                acc, rowidx, colidx, buf0, buf1, stage,
                sg0, sg1, ss0, ss1, sidx, sz0, sz1, sz2, sz3):
    c = lax.axis_index("c")
    s = lax.axis_index("s")
    wid = c * NS + s
    bufs = (buf0, buf1)
    sgs = (sg0, sg1)
    sss = (ss0, ss1)

    di_r = pltpu.async_copy(row_hbm.at[wid], rowidx, sidx)
    di_c = pltpu.async_copy(col_hbm.at[wid], colidx, sidx)

    _fill2d(stage, STG, D, 0.0)
    di_r.wait()
    di_c.wait()

    pltpu.async_copy(p_hbm.at[rowidx.at[0]], bufs[0], sgs[0])
    pltpu.async_copy(p_hbm.at[rowidx.at[1]], bufs[1], sgs[1])

    zsems = (sz0, sz1, sz2, sz3)

    def zslice(k):
        return acc.at[pl.ds(s * ROWS_PER_SUB + k * STG, STG)]

    def zbody(i, carry):
        k = 4 * i
        for j in range(4):
            pltpu.async_copy(stage, zslice(k + j), zsems[j])
        for j in range(4):
            pltpu.make_async_copy(stage, zslice(k + j), zsems[j]).wait()
        return carry

    lax.fori_loop(0, NSTG // 4, zbody, 0)
    pltpu.sync_copy(stage, zslice(NSTG - 1))
    plsc.subcore_barrier()

    def chunk(k, b):
        buf, sg, ss = bufs[b], sgs[b], sss[b]
        pltpu.make_async_copy(p_hbm.at[rowidx.at[k]], buf, sg).wait()
        pltpu.async_copy(buf, acc.at[colidx.at[k]], ss, add=True)
        pltpu.make_async_copy(buf, acc.at[colidx.at[k]], ss).wait()

        @pl.when(k + 2 < ITERS)
        def _next_gather():
            pltpu.async_copy(p_hbm.at[rowidx.at[k + 2]], buf, sg)

    def loop_body(k2, carry):
        chunk(2 * k2, 0)
        chunk(2 * k2 + 1, 1)
        return carry

    lax.fori_loop(0, ITERS // 2, loop_body, 0)
    chunk(ITERS - 1, 0)
    plsc.subcore_barrier()

    base = s * ROWS_PER_SUB
    pltpu.sync_copy(acc.at[pl.ds(base, ROWS_PER_SUB)],
                    g_out.at[c].at[pl.ds(base, ROWS_PER_SUB)])


BN = 2000


def _prep_body(degp_ref, x_ref, dis_ref, p_ref):
    deg = degp_ref[0] + degp_ref[1] + 1.0
    dis = lax.rsqrt(deg)
    dis_ref[...] = dis
    p_ref[...] = dis * x_ref[...]


def _prep(degp2, x):
    return pl.pallas_call(
        _prep_body,
        grid=(N // BN,),
        in_specs=[
            pl.BlockSpec((2, BN, 1), lambda i: (0, i, 0)),
            pl.BlockSpec((BN, D), lambda i: (i, 0)),
        ],
        out_specs=(pl.BlockSpec((BN, 1), lambda i: (i, 0)),
                   pl.BlockSpec((BN, D), lambda i: (i, 0))),
        out_shape=(jax.ShapeDtypeStruct((N, 1), jnp.float32),
                   jax.ShapeDtypeStruct((N, D), jnp.float32)),
    )(degp2, x)


def _tc_body(variant, *refs):
    if variant == "res":
        dis_ref, g_ref, p_ref, w_ref, b_ref, hres_ref, outh_ref, outp_ref = refs
    elif variant == "relu":
        dis_ref, g_ref, p_ref, w_ref, b_ref, outh_ref, outp_ref = refs
    elif variant == "mid":
        dis_ref, g_ref, p_ref, w_ref, b_ref, outp_ref = refs
    else:
        dis_ref, g_ref, p_ref, w_ref, b_ref, out_ref = refs
    dis = dis_ref[...]
    z = dis * (g_ref[0] + g_ref[1] + p_ref[...])
    y = jnp.dot(z, w_ref[...], preferred_element_type=jnp.float32) + b_ref[...]
    if variant == "plain":
        out_ref[...] = y
        return
    if variant == "mid":
        outp_ref[...] = dis * jnp.maximum(y, 0.0)
        return
    if variant == "relu":
        hnew = jnp.maximum(y, 0.0)
    else:
        hnew = (hres_ref[...] + jnp.maximum(y, 0.0)) * 0.5
    outh_ref[...] = hnew
    outp_ref[...] = dis * hnew


def _tc_layer(variant, dis2, g, p, w, b, hres=None):
    in_specs = [
        pl.BlockSpec((BN, 1), lambda i: (i, 0)),
        pl.BlockSpec((2, BN, D), lambda i: (0, i, 0)),
        pl.BlockSpec((BN, D), lambda i: (i, 0)),
        pl.BlockSpec((D, D), lambda i: (0, 0)),
        pl.BlockSpec((1, D), lambda i: (0, 0)),
    ]
    args = [dis2, g, p, w, b.reshape(1, D)]
    if variant == "res":
        in_specs.append(pl.BlockSpec((BN, D), lambda i: (i, 0)))
        args.append(hres)
    if variant in ("plain", "mid"):
        out_shape = jax.ShapeDtypeStruct((N, D), jnp.float32)
        out_specs = pl.BlockSpec((BN, D), lambda i: (i, 0))
    else:
        out_shape = (jax.ShapeDtypeStruct((N, D), jnp.float32),
                     jax.ShapeDtypeStruct((N, D), jnp.float32))
        out_specs = (pl.BlockSpec((BN, D), lambda i: (i, 0)),
                     pl.BlockSpec((BN, D), lambda i: (i, 0)))
    return pl.pallas_call(
        functools.partial(_tc_body, variant),
        grid=(N // BN,),
        in_specs=in_specs,
        out_specs=out_specs,
        out_shape=out_shape,
    )(*args)


def kernel(x, edge_index, weights, biases):
    row = edge_index[0]
    col = edge_index[1]
    row3 = row.reshape(NW, ITERS, CH)
    col3 = col.reshape(NW, ITERS, CH)

    degp = _deg_kernel(col3)
    dis2, p = _prep(degp.reshape(NC, N, 1), x)

    def agg(p):
        return _agg_kernel(p, row3, col3)

    h, p = _tc_layer("relu", dis2, agg(p), p, weights[0], biases[0])
    wi = 1
    for _ in range(3):
        pt = _tc_layer("mid", dis2, agg(p), p, weights[wi], biases[wi])
        h, p = _tc_layer("res", dis2, agg(pt), pt, weights[wi + 1],
                         biases[wi + 1], hres=h)
        wi += 2
    out = _tc_layer("plain", dis2, agg(p), p, weights[wi], biases[wi])
    return (out, h)
